# baseline (device time: 81261 ns/iter reference)
import os

import jax
import jax.numpy as jnp
from jax import lax
from jax.experimental import pallas as pl
from jax.experimental.pallas import tpu as pltpu

N_DEV = 4
N_Q = 4

_KMODE = os.environ.get("KMODE", "full")
_DO_COMM = _KMODE in ("full", "comm")
_DO_COMPUTE = _KMODE in ("full", "compute")


def kernel(x, w_mat):
    m_glob, k_per = x.shape
    k_glob, n = w_mat.shape
    m_blk = m_glob // N_DEV
    m_q = m_blk // N_Q

    def body(x_ref, w_ref, out_ref, xbf_ref, comm_ref, wstage_ref,
             xstage_ref, send_sems, recv_sems, qrecv_sems, dummy_sem,
             w_sems, x_sems):
        my = lax.axis_index("i")
        right = (my + 1) % N_DEV
        left = (my + 3) % N_DEV
        diag = (my + 2) % N_DEV

        def x_copy(t, slot):
            return pltpu.make_async_copy(
                x_ref.at[pl.ds(t * m_blk, m_blk), :],
                xstage_ref.at[slot],
                x_sems.at[slot],
            )

        def w_copy(src, slot):
            return pltpu.make_async_copy(
                w_ref.at[pl.ds(src * k_per, k_per), :],
                wstage_ref.at[slot],
                w_sems.at[slot],
            )

        x_copy(right, 0).start()
        x_copy(left, 1).start()
        w_copy(my, 0).start()
        w_copy(right, 1).start()

        if _DO_COMM:
            barrier_sem = pltpu.get_barrier_semaphore()
            for d in range(1, N_DEV):
                pl.semaphore_signal(
                    barrier_sem, inc=1,
                    device_id=((my + d) % N_DEV,),
                    device_id_type=pl.DeviceIdType.MESH,
                )
            pl.semaphore_wait(barrier_sem, N_DEV - 1)

        def cast_block(t, slot):
            xbf_ref[pl.ds(t * m_blk, m_blk), :] = (
                xstage_ref[slot].astype(jnp.bfloat16)
            )

        def send_block(t, sem_i):
            r = pltpu.make_async_remote_copy(
                src_ref=xbf_ref.at[pl.ds(t * m_blk, m_blk), :],
                dst_ref=comm_ref.at[my],
                send_sem=send_sems.at[sem_i],
                recv_sem=recv_sems.at[my],
                device_id=(t,),
                device_id_type=pl.DeviceIdType.MESH,
            )
            r.start()
            return r

        x_copy(right, 0).wait()
        cast_block(right, 0)
        s_r = send_block(right, 0) if _DO_COMM else None
        x_copy(diag, 0).start()

        x_copy(left, 1).wait()
        cast_block(left, 1)
        s_l = send_block(left, 1) if _DO_COMM else None
        x_copy(my, 1).start()

        x_copy(diag, 0).wait()
        cast_block(diag, 0)
        q_sends = []
        if _DO_COMM:
            for q in range(N_Q):
                r = pltpu.make_async_remote_copy(
                    src_ref=xbf_ref.at[
                        pl.ds(diag * m_blk + q * m_q, m_q), :
                    ],
                    dst_ref=comm_ref.at[my, pl.ds(q * m_q, m_q), :],
                    send_sem=send_sems.at[2 + q],
                    recv_sem=qrecv_sems.at[q],
                    device_id=(diag,),
                    device_id_type=pl.DeviceIdType.MESH,
                )
                r.start()
                q_sends.append(r)

        x_copy(my, 1).wait()
        w_copy(my, 0).wait()
        if _DO_COMPUTE:
            out_ref[...] = jnp.dot(
                xstage_ref[1].astype(jnp.bfloat16),
                wstage_ref[0].astype(jnp.bfloat16),
                preferred_element_type=jnp.float32,
            )
        else:
            out_ref[...] = jnp.zeros((m_blk, n), jnp.float32)

        def wait_recv_from(src):
            recv = pltpu.make_async_remote_copy(
                src_ref=comm_ref.at[src],
                dst_ref=comm_ref.at[src],
                send_sem=dummy_sem.at[0],
                recv_sem=recv_sems.at[src],
                device_id=(src,),
                device_id_type=pl.DeviceIdType.MESH,
            )
            recv.wait_recv()

        for src, slot, nxt in [(right, 1, (left, 0)), (left, 0, (diag, 1))]:
            if _DO_COMM:
                wait_recv_from(src)
            w_copy(src, slot).wait()
            w_copy(nxt[0], nxt[1]).start()
            if _DO_COMPUTE:
                out_ref[...] += jnp.dot(
                    comm_ref[src],
                    wstage_ref[slot].astype(jnp.bfloat16),
                    preferred_element_type=jnp.float32,
                )
            else:
                out_ref[0:1, 0:1] += (
                    comm_ref[src][0:1, 0:1].astype(jnp.float32)
                )

        w_copy(diag, 1).wait()
        for q in range(N_Q):
            if _DO_COMM:
                recv = pltpu.make_async_remote_copy(
                    src_ref=comm_ref.at[diag, pl.ds(q * m_q, m_q), :],
                    dst_ref=comm_ref.at[diag, pl.ds(q * m_q, m_q), :],
                    send_sem=dummy_sem.at[0],
                    recv_sem=qrecv_sems.at[q],
                    device_id=(diag,),
                    device_id_type=pl.DeviceIdType.MESH,
                )
                recv.wait_recv()
            if _DO_COMPUTE:
                rows = pl.ds(q * m_q, m_q)
                y = out_ref[rows, :] + jnp.dot(
                    comm_ref[diag, pl.ds(q * m_q, m_q), :],
                    wstage_ref[1].astype(jnp.bfloat16),
                    preferred_element_type=jnp.float32,
                )
                out_ref[rows, :] = y * (1.0 / (1.0 + jnp.exp(-y)))

        if _DO_COMM:
            s_r.wait_send()
            s_l.wait_send()
        for r in q_sends:
            r.wait_send()

    return pl.pallas_call(
        body,
        out_shape=jax.ShapeDtypeStruct((m_blk, n), jnp.float32),
        in_specs=[
            pl.BlockSpec(memory_space=pl.ANY),
            pl.BlockSpec(memory_space=pl.ANY),
        ],
        out_specs=pl.BlockSpec(memory_space=pltpu.VMEM),
        scratch_shapes=[
            pltpu.VMEM((m_glob, k_per), jnp.bfloat16),
            pltpu.VMEM((N_DEV, m_blk, k_per), jnp.bfloat16),
            pltpu.VMEM((2, k_per, n), jnp.float32),
            pltpu.VMEM((2, m_blk, k_per), jnp.float32),
            pltpu.SemaphoreType.DMA((2 + N_Q,)),
            pltpu.SemaphoreType.DMA((N_DEV,)),
            pltpu.SemaphoreType.DMA((N_Q,)),
            pltpu.SemaphoreType.DMA((1,)),
            pltpu.SemaphoreType.DMA((2,)),
            pltpu.SemaphoreType.DMA((2,)),
        ],
        compiler_params=pltpu.CompilerParams(
            collective_id=0 if _DO_COMM else None,
            vmem_limit_bytes=60 * 1024 * 1024,
        ),
    )(x, w_mat)


# device time: 69372 ns/iter; 1.1714x vs baseline; 1.1714x over previous
import os

import jax
import jax.numpy as jnp
from jax import lax
from jax.experimental import pallas as pl
from jax.experimental.pallas import tpu as pltpu

N_DEV = 4
N_Q = 8

_KMODE = os.environ.get("KMODE", "full")
_DO_COMM = _KMODE in ("full", "comm")
_DO_COMPUTE = _KMODE in ("full", "compute")


def kernel(x, w_mat):
    m_glob, k_per = x.shape
    k_glob, n = w_mat.shape
    m_blk = m_glob // N_DEV
    m_q = m_blk // N_Q

    def body(x_ref, w_ref, out_ref, xbf_ref, comm_ref, wstage_ref,
             send_sems, recv_sems, qrecv_sems, dummy_sem, w_sems):
        my = lax.axis_index("i")
        right = (my + 1) % N_DEV
        left = (my + 3) % N_DEV
        diag = (my + 2) % N_DEV

        if _DO_COMM:
            barrier_sem = pltpu.get_barrier_semaphore()
            for d in range(1, N_DEV):
                pl.semaphore_signal(
                    barrier_sem, inc=1,
                    device_id=((my + d) % N_DEV,),
                    device_id_type=pl.DeviceIdType.MESH,
                )
            pl.semaphore_wait(barrier_sem, N_DEV - 1)

        def w_copy(src, slot):
            return pltpu.make_async_copy(
                w_ref.at[pl.ds(src * k_per, k_per), :],
                wstage_ref.at[slot],
                w_sems.at[slot],
            )

        w_copy(my, 0).start()
        w_copy(right, 1).start()

        def cast_block(t):
            xbf_ref[pl.ds(t * m_blk, m_blk), :] = (
                x_ref[pl.ds(t * m_blk, m_blk), :].astype(jnp.bfloat16)
            )

        def send_block(t, sem_i):
            r = pltpu.make_async_remote_copy(
                src_ref=xbf_ref.at[pl.ds(t * m_blk, m_blk), :],
                dst_ref=comm_ref.at[my],
                send_sem=send_sems.at[sem_i],
                recv_sem=recv_sems.at[my],
                device_id=(t,),
                device_id_type=pl.DeviceIdType.MESH,
            )
            r.start()
            return r

        cast_block(right)
        s_r = send_block(right, 0) if _DO_COMM else None
        cast_block(left)
        s_l = send_block(left, 1) if _DO_COMM else None
        cast_block(diag)

        q_sends = []
        if _DO_COMM:
            for q in range(N_Q):
                r = pltpu.make_async_remote_copy(
                    src_ref=xbf_ref.at[
                        pl.ds(diag * m_blk + q * m_q, m_q), :
                    ],
                    dst_ref=comm_ref.at[my, pl.ds(q * m_q, m_q), :],
                    send_sem=send_sems.at[2 + q],
                    recv_sem=qrecv_sems.at[q],
                    device_id=(diag,),
                    device_id_type=pl.DeviceIdType.MESH,
                )
                r.start()
                q_sends.append(r)

        w_copy(my, 0).wait()
        if _DO_COMPUTE:
            out_ref[...] = jnp.dot(
                x_ref[pl.ds(my * m_blk, m_blk), :].astype(jnp.bfloat16),
                wstage_ref[0].astype(jnp.bfloat16),
                preferred_element_type=jnp.float32,
            )
        else:
            out_ref[...] = jnp.zeros((m_blk, n), jnp.float32)

        def wait_recv_from(src):
            recv = pltpu.make_async_remote_copy(
                src_ref=comm_ref.at[src],
                dst_ref=comm_ref.at[src],
                send_sem=dummy_sem.at[0],
                recv_sem=recv_sems.at[src],
                device_id=(src,),
                device_id_type=pl.DeviceIdType.MESH,
            )
            recv.wait_recv()

        for src, slot, nxt in [(right, 1, (left, 0)), (left, 0, (diag, 1))]:
            if _DO_COMM:
                wait_recv_from(src)
            w_copy(src, slot).wait()
            w_copy(nxt[0], nxt[1]).start()
            if _DO_COMPUTE:
                out_ref[...] += jnp.dot(
                    comm_ref[src],
                    wstage_ref[slot].astype(jnp.bfloat16),
                    preferred_element_type=jnp.float32,
                )
            else:
                out_ref[0:1, 0:1] += (
                    comm_ref[src][0:1, 0:1].astype(jnp.float32)
                )

        w_copy(diag, 1).wait()
        for q in range(N_Q):
            if _DO_COMM:
                recv = pltpu.make_async_remote_copy(
                    src_ref=comm_ref.at[diag, pl.ds(q * m_q, m_q), :],
                    dst_ref=comm_ref.at[diag, pl.ds(q * m_q, m_q), :],
                    send_sem=dummy_sem.at[0],
                    recv_sem=qrecv_sems.at[q],
                    device_id=(diag,),
                    device_id_type=pl.DeviceIdType.MESH,
                )
                recv.wait_recv()
            if _DO_COMPUTE:
                rows = pl.ds(q * m_q, m_q)
                y = out_ref[rows, :] + jnp.dot(
                    comm_ref[diag, pl.ds(q * m_q, m_q), :],
                    wstage_ref[1].astype(jnp.bfloat16),
                    preferred_element_type=jnp.float32,
                )
                out_ref[rows, :] = y * (1.0 / (1.0 + jnp.exp(-y)))

        if _DO_COMM:
            s_r.wait_send()
            s_l.wait_send()
        for r in q_sends:
            r.wait_send()

    return pl.pallas_call(
        body,
        out_shape=jax.ShapeDtypeStruct((m_blk, n), jnp.float32),
        in_specs=[
            pl.BlockSpec(memory_space=pltpu.VMEM),
            pl.BlockSpec(memory_space=pl.ANY),
        ],
        out_specs=pl.BlockSpec(memory_space=pltpu.VMEM),
        scratch_shapes=[
            pltpu.VMEM((m_glob, k_per), jnp.bfloat16),
            pltpu.VMEM((N_DEV, m_blk, k_per), jnp.bfloat16),
            pltpu.VMEM((2, k_per, n), jnp.float32),
            pltpu.SemaphoreType.DMA((2 + N_Q,)),
            pltpu.SemaphoreType.DMA((N_DEV,)),
            pltpu.SemaphoreType.DMA((N_Q,)),
            pltpu.SemaphoreType.DMA((1,)),
            pltpu.SemaphoreType.DMA((2,)),
        ],
        compiler_params=pltpu.CompilerParams(
            collective_id=0 if _DO_COMM else None,
            vmem_limit_bytes=60 * 1024 * 1024,
        ),
    )(x, w_mat)
